# baseline (speedup 1.0000x reference)
import functools

import jax
import jax.numpy as jnp
from jax import lax
from jax.experimental import pallas as pl
from jax.experimental.pallas import tpu as pltpu
from jax.experimental.pallas import tpu_sc as plsc

N = 100000
E = 1600000
B = 512
H = 32
NP = 100352
ER = E // 128
RPT = NP // 16
GRP = 4
CH1 = 7168
CH2 = 3584


def _sc_edge_body(stage_a, e3, ta, tb, zz, o0, o1, sidx, didx, rows,
                  gsem, ssem, isem, acc):
    c = lax.axis_index("c")
    s = lax.axis_index("s")

    def fire_idx(q, rb):
        pltpu.async_copy(e3.at[0, pl.ds(rb, GRP)],
                         sidx.at[q, pl.ds(0, GRP)], isem)
        pltpu.async_copy(e3.at[1, pl.ds(rb, GRP)],
                         didx.at[q, pl.ds(0, GRP)], isem)

    def wait_idx(q, rb):
        pltpu.make_async_copy(e3.at[0, pl.ds(rb, GRP)],
                              sidx.at[q, pl.ds(0, GRP)], isem).wait()
        pltpu.make_async_copy(e3.at[1, pl.ds(rb, GRP)],
                              didx.at[q, pl.ds(0, GRP)], isem).wait()

    def drain_scatters(rset, q):
        for j in range(GRP):
            pltpu.make_async_copy(rows.at[rset, j],
                                  acc.at[didx.at[q, j]], ssem).wait()

    def sync_group(tbl, rb, nr):
        pltpu.sync_copy(e3.at[0, pl.ds(rb, nr)], sidx.at[0, pl.ds(0, nr)])
        pltpu.sync_copy(e3.at[1, pl.ds(rb, nr)], didx.at[0, pl.ds(0, nr)])
        for j in range(nr):
            pltpu.async_copy(tbl.at[sidx.at[0, j]], rows.at[0, j], gsem)
        for j in range(nr):
            pltpu.make_async_copy(tbl.at[sidx.at[0, j]], rows.at[0, j],
                                  gsem).wait()
            pltpu.async_copy(rows.at[0, j], acc.at[didx.at[0, j]], ssem,
                             add=True)
        for j in range(nr):
            pltpu.make_async_copy(rows.at[0, j], acc.at[didx.at[0, j]],
                                  ssem).wait()

    def work(tbl, out):
        pltpu.sync_copy(zz.at[pl.ds(s * RPT, RPT)],
                        acc.at[pl.ds(s * RPT, RPT)])
        plsc.subcore_barrier()
        if stage_a:
            w = c * 16 + s
            r0 = (ER * w) // 32
            r1 = (ER * (w + 1)) // 32
        else:
            r0 = (ER * s) // 16
            r1 = (ER * (s + 1)) // 16
        nq = (r1 - r0) // (4 * GRP)
        t0 = r0 + nq * 4 * GRP

        fire_idx(0, r0)
        fire_idx(1, r0 + GRP)

        def quad(i, _):
            rb = r0 + i * 4 * GRP
            for g in range(4):
                rset = g % 2
                q = g
                qn = (g + 2) % 4
                if g < 2:
                    pl.when(i > 0)(
                        functools.partial(drain_scatters, rset, qn))
                else:
                    drain_scatters(rset, qn)
                wait_idx(q, rb + g * GRP)
                for j in range(GRP):
                    pltpu.async_copy(tbl.at[sidx.at[q, j]],
                                     rows.at[rset, j], gsem)
                nrb = rb + (g + 2) * GRP
                pl.when(nrb < t0)(functools.partial(fire_idx, qn, nrb))
                for j in range(GRP):
                    pltpu.make_async_copy(tbl.at[sidx.at[q, j]],
                                          rows.at[rset, j], gsem).wait()
                    pltpu.async_copy(rows.at[rset, j], acc.at[didx.at[q, j]],
                                     ssem, add=True)
            return 0
        lax.fori_loop(0, nq, quad, 0)
        drain_scatters(0, 2)
        drain_scatters(1, 3)

        def tail(r, _):
            sync_group(tbl, r, 1)
            return 0
        lax.fori_loop(t0, r1, tail, 0)
        plsc.subcore_barrier()
        pltpu.sync_copy(acc.at[pl.ds(s * RPT, RPT)],
                        out.at[pl.ds(s * RPT, RPT)])

    pl.when(c == 0)(lambda: work(ta, o0))
    pl.when(c == 1)(lambda: work(tb, o1))


def _sc_edge_agg(stage_a, e3, ta, tb):
    width = 16
    mesh = plsc.VectorSubcoreMesh(core_axis_name="c", subcore_axis_name="s")
    out = [jax.ShapeDtypeStruct((NP, width), jnp.float32)] * 2
    scratch = [
        pltpu.VMEM((4, GRP, 128), jnp.int32),
        pltpu.VMEM((4, GRP, 128), jnp.int32),
        pltpu.VMEM((2, GRP, 128, width), jnp.float32),
        pltpu.SemaphoreType.DMA,
        pltpu.SemaphoreType.DMA,
        pltpu.SemaphoreType.DMA,
        pltpu.VMEM_SHARED((NP, width), jnp.float32),
    ]
    zz = jnp.zeros((NP, width), jnp.float32)
    f = pl.kernel(functools.partial(_sc_edge_body, stage_a),
                  out_type=out, mesh=mesh, scratch_types=scratch,
                  compiler_params=pltpu.CompilerParams(
                      use_tc_tiling_on_sc=False))
    return f(e3, ta, tb, zz)


def _b1_body(pa_ref, pb_ref, x_ref, g_ref):
    i = pl.program_id(0)
    tot = pa_ref[...] + pb_ref[...]
    cnt = tot[:, 1:2]
    mean = tot[:, 0:1] / jnp.maximum(cnt, 1.0)
    row = i * CH1 + lax.broadcasted_iota(jnp.int32, (CH1, 1), 0)
    valid = (row < N).astype(jnp.float32)
    z = jnp.concatenate([mean, x_ref[...], jnp.ones((CH1, 1), jnp.float32)],
                        axis=1) * valid
    zz = lax.dot_general(z, z, (((0,), (0,)), ((), ())),
                         preferred_element_type=jnp.float32)

    @pl.when(i == 0)
    def _():
        g_ref[...] = jnp.zeros_like(g_ref)

    g_ref[...] += zz


def _b2_body(pa_ref, pb_ref, x_ref, c_ref, ha_ref, hb_ref):
    tot = pa_ref[...] + pb_ref[...]
    mean = tot[:, 0:1] / jnp.maximum(tot[:, 1:2], 1.0)
    z = jnp.concatenate([mean, x_ref[...],
                         jnp.ones((CH1, 1), jnp.float32)], axis=1)
    h1 = jax.nn.relu(jnp.dot(z, c_ref[...],
                             preferred_element_type=jnp.float32))
    ha_ref[...] = h1[:, :16]
    hb_ref[...] = h1[:, 16:]


def _d1_body(sa_ref, sb_ref, pa_ref, pb_ref, ha_ref, hb_ref, wl_ref, wr_ref,
             bias_ref, pre_ref, st_ref):
    i = pl.program_id(0)
    cnt = pa_ref[:, 1:2] + pb_ref[:, 1:2]
    s2 = jnp.concatenate([sa_ref[...], sb_ref[...]], axis=1)
    mean2 = s2 / jnp.maximum(cnt, 1.0)
    h1 = jnp.concatenate([ha_ref[...], hb_ref[...]], axis=1)
    pre = (jnp.dot(mean2, wl_ref[...], preferred_element_type=jnp.float32)
           + jnp.dot(h1, wr_ref[...], preferred_element_type=jnp.float32)
           + bias_ref[...])
    pre_ref[...] = pre
    row = i * CH1 + lax.broadcasted_iota(jnp.int32, (CH1, 1), 0)
    prem = jnp.where(row < N, pre, 0.0)

    @pl.when(i == 0)
    def _():
        st_ref[...] = jnp.zeros_like(st_ref)

    st_ref[0:1, :] += jnp.sum(prem, axis=0, keepdims=True)
    st_ref[1:2, :] += jnp.sum(prem * prem, axis=0, keepdims=True)


def _d23_body(pre_ref, b_ref, sc_ref, sh_ref, wg_ref, bg_ref, m_ref, den_ref,
              num_ref):
    i = pl.program_id(0)
    h2 = jax.nn.relu(pre_ref[...] * sc_ref[...] + sh_ref[...])
    lg = jnp.sum(h2 * wg_ref[...], axis=1, keepdims=True) + bg_ref[0, 0]
    row = i * CH2 + lax.broadcasted_iota(jnp.int32, (CH2, 1), 0)
    lg = jnp.where(row < N, lg, -1e30)

    @pl.when(i == 0)
    def _():
        m_ref[...] = jnp.full_like(m_ref, -1e30)
        den_ref[...] = jnp.zeros_like(den_ref)
        num_ref[...] = jnp.zeros_like(num_ref)

    m_old = m_ref[0, 0]
    m_new = jnp.maximum(m_old, jnp.max(lg))
    r = jnp.exp(m_old - m_new)
    m_ref[...] = jnp.full((1, 1), m_new, jnp.float32)
    ex = jnp.exp(lg - m_new)
    bio = lax.broadcasted_iota(jnp.int32, (CH2, B), 1)
    w = jnp.where(bio == b_ref[...], ex, 0.0)
    den_ref[...] = den_ref[...] * r + jnp.sum(w, axis=0, keepdims=True)
    num_ref[...] = num_ref[...] * r + lax.dot_general(
        w, h2, (((0,), (0,)), ((), ())), preferred_element_type=jnp.float32)


def _d4_body(num_ref, den_ref, di_ref, ci_ref, dt_ref, ct_ref, w1_ref,
             b1_ref, w2_ref, b2_ref, o_ref):
    den = den_ref[...]
    g = jnp.where(den > 0.0, num_ref[...] / den, 0.0)
    iod = lax.broadcasted_iota(jnp.int32, (B, 1536), 1)
    ohd = (iod == di_ref[...]).astype(jnp.float32)
    ed = jnp.dot(ohd, dt_ref[...], preferred_element_type=jnp.float32)
    ioc = lax.broadcasted_iota(jnp.int32, (B, 1024), 1)
    ohc = (ioc == ci_ref[...]).astype(jnp.float32)
    ec = jnp.dot(ohc, ct_ref[...], preferred_element_type=jnp.float32)
    hcat = jnp.concatenate([g, ed, ec], axis=1)
    hh = jax.nn.relu(jnp.dot(hcat, w1_ref[...],
                             preferred_element_type=jnp.float32) + b1_ref[...])
    out = jnp.dot(hh, w2_ref[...], preferred_element_type=jnp.float32)
    o_ref[...] = jax.nn.sigmoid(out + b2_ref[0, 0])


def _chunk_spec(ch, w):
    return pl.BlockSpec((ch, w), lambda i: (i, 0))


def _full_spec(shape):
    return pl.BlockSpec(shape, lambda i: tuple(0 for _ in shape))


def kernel(x, edge_index, batch, drug_idx, cell_idx, Wl1, bl1, Wr1, br1, g1,
           be1, Wl2, bl2, Wr2, br2, g2, be2, Wg, bg, drug_table, cell_table,
           Wh1, bh1, Wh2, bh2):
    f32 = jnp.float32
    e3 = edge_index.reshape(2, ER, 128)
    xp = jnp.pad(x, ((0, NP - N), (0, 0)))
    xo16 = jnp.concatenate(
        [xp, jnp.ones((NP, 1), f32), jnp.zeros((NP, 14), f32)], axis=1)
    batch_p = jnp.pad(batch, (0, NP - N), constant_values=-1).reshape(NP, 1)

    pa, pb = _sc_edge_agg(True, e3, xo16, xo16)

    gmat = pl.pallas_call(
        _b1_body,
        grid=(NP // CH1,),
        in_specs=[_chunk_spec(CH1, 16), _chunk_spec(CH1, 16),
                  _chunk_spec(CH1, 1)],
        out_specs=_full_spec((3, 3)),
        out_shape=jax.ShapeDtypeStruct((3, 3), f32),
    )(pa, pb, xp)

    wl = Wl1[:, 0]
    wr = Wr1[:, 0]
    c0 = bl1 + br1
    sm, sx = gmat[0, 2], gmat[1, 2]
    smm, sxx, smx = gmat[0, 0], gmat[1, 1], gmat[0, 1]
    mu = (sm * wl + sx * wr) / N + c0
    e2 = (smm * wl * wl + sxx * wr * wr + 2.0 * smx * wl * wr
          + 2.0 * c0 * (sm * wl + sx * wr)) / N + c0 * c0
    var = e2 - mu * mu
    inv = g1 / jnp.sqrt(var + 1e-5)
    cmat = jnp.stack([inv * wl, inv * wr, inv * (c0 - mu) + be1])

    h1a, h1b = pl.pallas_call(
        _b2_body,
        grid=(NP // CH1,),
        in_specs=[_chunk_spec(CH1, 16), _chunk_spec(CH1, 16),
                  _chunk_spec(CH1, 1), _full_spec((3, H))],
        out_specs=[_chunk_spec(CH1, 16), _chunk_spec(CH1, 16)],
        out_shape=[jax.ShapeDtypeStruct((NP, 16), f32)] * 2,
    )(pa, pb, xp, cmat)

    s2a, s2b = _sc_edge_agg(False, e3, h1a, h1b)

    pre2, stats = pl.pallas_call(
        _d1_body,
        grid=(NP // CH1,),
        in_specs=[_chunk_spec(CH1, 16)] * 4
        + [_chunk_spec(CH1, 16)] * 2
        + [_full_spec((H, H)), _full_spec((H, H)), _full_spec((1, H))],
        out_specs=[_chunk_spec(CH1, H), _full_spec((2, H))],
        out_shape=[jax.ShapeDtypeStruct((NP, H), f32),
                   jax.ShapeDtypeStruct((2, H), f32)],
    )(s2a, s2b, pa, pb, h1a, h1b, Wl2.T, Wr2.T, (bl2 + br2).reshape(1, H))

    m2 = stats[0] / N
    v2 = stats[1] / N - m2 * m2
    inv2 = g2 / jnp.sqrt(v2 + 1e-5)
    sc2 = inv2.reshape(1, H)
    sh2 = (be2 - m2 * inv2).reshape(1, H)

    _, den, num = pl.pallas_call(
        _d23_body,
        grid=(NP // CH2,),
        in_specs=[_chunk_spec(CH2, H), _chunk_spec(CH2, 1),
                  _full_spec((1, H)), _full_spec((1, H)),
                  _full_spec((1, H)), _full_spec((1, 1))],
        out_specs=[_full_spec((1, 1)), _full_spec((1, B)),
                   _full_spec((B, H))],
        out_shape=[jax.ShapeDtypeStruct((1, 1), f32),
                   jax.ShapeDtypeStruct((1, B), f32),
                   jax.ShapeDtypeStruct((B, H), f32)],
    )(pre2, batch_p, sc2, sh2, Wg, bg.reshape(1, 1))

    dtp = jnp.pad(drug_table, ((0, 1536 - drug_table.shape[0]), (0, 0)))
    ctp = jnp.pad(cell_table, ((0, 1024 - cell_table.shape[0]), (0, 0)))
    w2p = jnp.zeros((H, 128), f32).at[:, 0].set(Wh2[0])
    out = pl.pallas_call(
        _d4_body,
        in_specs=[_full_spec((B, H)), _full_spec((B, 1)),
                  _full_spec((B, 1)), _full_spec((B, 1)),
                  _full_spec((1536, 16)), _full_spec((1024, 16)),
                  _full_spec((2 * 16 + H, H)), _full_spec((1, H)),
                  _full_spec((H, 128)), _full_spec((1, 1))],
        out_specs=_full_spec((B, 128)),
        out_shape=jax.ShapeDtypeStruct((B, 128), f32),
        grid=(1,),
    )(num, den.reshape(B, 1), drug_idx.reshape(B, 1),
      cell_idx.reshape(B, 1), dtp, ctp, Wh1.T, bh1.reshape(1, H), w2p,
      bh2.reshape(1, 1))
    return out[:, 0]


# speedup vs baseline: 1.0489x; 1.0489x over previous
import functools

import jax
import jax.numpy as jnp
from jax import lax
from jax.experimental import pallas as pl
from jax.experimental.pallas import tpu as pltpu
from jax.experimental.pallas import tpu_sc as plsc

N = 100000
E = 1600000
B = 512
H = 32
NP = 100352
ER = E // 128
RPT = NP // 16
GRP = 5
CH1 = 7168
CH2 = 3584


def _sc_edge_body(stage_a, e3, ta, tb, o0, o1, sidx, didx, rows, zbuf,
                  gsem, ssem, isem, acc):
    c = lax.axis_index("c")
    s = lax.axis_index("s")

    def fire_idx(q, rb):
        pltpu.async_copy(e3.at[0, pl.ds(rb, GRP)],
                         sidx.at[q, pl.ds(0, GRP)], isem)
        pltpu.async_copy(e3.at[1, pl.ds(rb, GRP)],
                         didx.at[q, pl.ds(0, GRP)], isem)

    def wait_idx(q, rb):
        pltpu.make_async_copy(e3.at[0, pl.ds(rb, GRP)],
                              sidx.at[q, pl.ds(0, GRP)], isem).wait()
        pltpu.make_async_copy(e3.at[1, pl.ds(rb, GRP)],
                              didx.at[q, pl.ds(0, GRP)], isem).wait()

    def drain_scatters(rset, q):
        for j in range(GRP):
            pltpu.make_async_copy(rows.at[rset, j],
                                  acc.at[didx.at[q, j]], ssem).wait()

    def sync_group(tbl, rb, nr):
        pltpu.sync_copy(e3.at[0, pl.ds(rb, nr)], sidx.at[0, pl.ds(0, nr)])
        pltpu.sync_copy(e3.at[1, pl.ds(rb, nr)], didx.at[0, pl.ds(0, nr)])
        for j in range(nr):
            pltpu.async_copy(tbl.at[sidx.at[0, j]], rows.at[0, j], gsem)
        for j in range(nr):
            pltpu.make_async_copy(tbl.at[sidx.at[0, j]], rows.at[0, j],
                                  gsem).wait()
            pltpu.async_copy(rows.at[0, j], acc.at[didx.at[0, j]], ssem,
                             add=True)
        for j in range(nr):
            pltpu.make_async_copy(rows.at[0, j], acc.at[didx.at[0, j]],
                                  ssem).wait()

    def work(tbl, out):
        def zloop(i, _):
            zbuf[i, :] = jnp.zeros((16,), jnp.float32)
            return 0
        lax.fori_loop(0, 128, zloop, 0)
        for k in range(RPT // 128):
            pltpu.sync_copy(zbuf, acc.at[pl.ds(s * RPT + k * 128, 128)])
        plsc.subcore_barrier()
        if stage_a:
            w = c * 16 + s
            r0 = (ER * w) // 32
            r1 = (ER * (w + 1)) // 32
        else:
            r0 = (ER * s) // 16
            r1 = (ER * (s + 1)) // 16
        nq = (r1 - r0) // (4 * GRP)
        t0 = r0 + nq * 4 * GRP

        fire_idx(0, r0)
        fire_idx(1, r0 + GRP)

        def quad(i, _):
            rb = r0 + i * 4 * GRP
            for g in range(4):
                rset = g % 2
                q = g
                qn = (g + 2) % 4
                if g < 2:
                    pl.when(i > 0)(
                        functools.partial(drain_scatters, rset, qn))
                else:
                    drain_scatters(rset, qn)
                wait_idx(q, rb + g * GRP)
                for j in range(GRP):
                    pltpu.async_copy(tbl.at[sidx.at[q, j]],
                                     rows.at[rset, j], gsem)
                nrb = rb + (g + 2) * GRP
                pl.when(nrb < t0)(functools.partial(fire_idx, qn, nrb))
                for j in range(GRP):
                    pltpu.make_async_copy(tbl.at[sidx.at[q, j]],
                                          rows.at[rset, j], gsem).wait()
                    pltpu.async_copy(rows.at[rset, j], acc.at[didx.at[q, j]],
                                     ssem, add=True)
            return 0
        lax.fori_loop(0, nq, quad, 0)
        drain_scatters(0, 2)
        drain_scatters(1, 3)

        def tail(r, _):
            sync_group(tbl, r, 1)
            return 0
        lax.fori_loop(t0, r1, tail, 0)
        plsc.subcore_barrier()
        pltpu.sync_copy(acc.at[pl.ds(s * RPT, RPT)],
                        out.at[pl.ds(s * RPT, RPT)])

    pl.when(c == 0)(lambda: work(ta, o0))
    pl.when(c == 1)(lambda: work(tb, o1))


def _sc_edge_agg(stage_a, e3, ta, tb):
    width = 16
    mesh = plsc.VectorSubcoreMesh(core_axis_name="c", subcore_axis_name="s")
    out = [jax.ShapeDtypeStruct((NP, width), jnp.float32)] * 2
    scratch = [
        pltpu.VMEM((4, GRP, 128), jnp.int32),
        pltpu.VMEM((4, GRP, 128), jnp.int32),
        pltpu.VMEM((2, GRP, 128, width), jnp.float32),
        pltpu.VMEM((128, 16), jnp.float32),
        pltpu.SemaphoreType.DMA,
        pltpu.SemaphoreType.DMA,
        pltpu.SemaphoreType.DMA,
        pltpu.VMEM_SHARED((NP, width), jnp.float32),
    ]
    f = pl.kernel(functools.partial(_sc_edge_body, stage_a),
                  out_type=out, mesh=mesh, scratch_types=scratch,
                  compiler_params=pltpu.CompilerParams(
                      use_tc_tiling_on_sc=False))
    return f(e3, ta, tb)


def _b1_body(pa_ref, pb_ref, x_ref, g_ref):
    i = pl.program_id(0)
    tot = pa_ref[...] + pb_ref[...]
    cnt = tot[:, 1:2]
    mean = tot[:, 0:1] / jnp.maximum(cnt, 1.0)
    row = i * CH1 + lax.broadcasted_iota(jnp.int32, (CH1, 1), 0)
    valid = (row < N).astype(jnp.float32)
    z = jnp.concatenate([mean, x_ref[...], jnp.ones((CH1, 1), jnp.float32)],
                        axis=1) * valid
    zz = lax.dot_general(z, z, (((0,), (0,)), ((), ())),
                         preferred_element_type=jnp.float32)

    @pl.when(i == 0)
    def _():
        g_ref[...] = jnp.zeros_like(g_ref)

    g_ref[...] += zz


def _b2_body(pa_ref, pb_ref, x_ref, c_ref, ha_ref, hb_ref):
    tot = pa_ref[...] + pb_ref[...]
    mean = tot[:, 0:1] / jnp.maximum(tot[:, 1:2], 1.0)
    z = jnp.concatenate([mean, x_ref[...],
                         jnp.ones((CH1, 1), jnp.float32)], axis=1)
    h1 = jax.nn.relu(jnp.dot(z, c_ref[...],
                             preferred_element_type=jnp.float32))
    ha_ref[...] = h1[:, :16]
    hb_ref[...] = h1[:, 16:]


def _d1_body(sa_ref, sb_ref, pa_ref, pb_ref, ha_ref, hb_ref, wl_ref, wr_ref,
             bias_ref, pre_ref, st_ref):
    i = pl.program_id(0)
    cnt = pa_ref[:, 1:2] + pb_ref[:, 1:2]
    s2 = jnp.concatenate([sa_ref[...], sb_ref[...]], axis=1)
    mean2 = s2 / jnp.maximum(cnt, 1.0)
    h1 = jnp.concatenate([ha_ref[...], hb_ref[...]], axis=1)
    pre = (jnp.dot(mean2, wl_ref[...], preferred_element_type=jnp.float32)
           + jnp.dot(h1, wr_ref[...], preferred_element_type=jnp.float32)
           + bias_ref[...])
    pre_ref[...] = pre
    row = i * CH1 + lax.broadcasted_iota(jnp.int32, (CH1, 1), 0)
    prem = jnp.where(row < N, pre, 0.0)

    @pl.when(i == 0)
    def _():
        st_ref[...] = jnp.zeros_like(st_ref)

    st_ref[0:1, :] += jnp.sum(prem, axis=0, keepdims=True)
    st_ref[1:2, :] += jnp.sum(prem * prem, axis=0, keepdims=True)


def _d23_body(pre_ref, b_ref, sc_ref, sh_ref, wg_ref, bg_ref, m_ref, den_ref,
              num_ref):
    i = pl.program_id(0)
    h2 = jax.nn.relu(pre_ref[...] * sc_ref[...] + sh_ref[...])
    lg = jnp.sum(h2 * wg_ref[...], axis=1, keepdims=True) + bg_ref[0, 0]
    row = i * CH2 + lax.broadcasted_iota(jnp.int32, (CH2, 1), 0)
    lg = jnp.where(row < N, lg, -1e30)

    @pl.when(i == 0)
    def _():
        m_ref[...] = jnp.full_like(m_ref, -1e30)
        den_ref[...] = jnp.zeros_like(den_ref)
        num_ref[...] = jnp.zeros_like(num_ref)

    m_old = m_ref[0, 0]
    m_new = jnp.maximum(m_old, jnp.max(lg))
    r = jnp.exp(m_old - m_new)
    m_ref[...] = jnp.full((1, 1), m_new, jnp.float32)
    ex = jnp.exp(lg - m_new)
    bio = lax.broadcasted_iota(jnp.int32, (CH2, B), 1)
    w = jnp.where(bio == b_ref[...], ex, 0.0)
    den_ref[...] = den_ref[...] * r + jnp.sum(w, axis=0, keepdims=True)
    num_ref[...] = num_ref[...] * r + lax.dot_general(
        w, h2, (((0,), (0,)), ((), ())), preferred_element_type=jnp.float32)


def _d4_body(num_ref, den_ref, di_ref, ci_ref, dt_ref, ct_ref, w1_ref,
             b1_ref, w2_ref, b2_ref, o_ref):
    den = den_ref[...]
    g = jnp.where(den > 0.0, num_ref[...] / den, 0.0)
    iod = lax.broadcasted_iota(jnp.int32, (B, 1536), 1)
    ohd = (iod == di_ref[...]).astype(jnp.float32)
    ed = jnp.dot(ohd, dt_ref[...], preferred_element_type=jnp.float32)
    ioc = lax.broadcasted_iota(jnp.int32, (B, 1024), 1)
    ohc = (ioc == ci_ref[...]).astype(jnp.float32)
    ec = jnp.dot(ohc, ct_ref[...], preferred_element_type=jnp.float32)
    hcat = jnp.concatenate([g, ed, ec], axis=1)
    hh = jax.nn.relu(jnp.dot(hcat, w1_ref[...],
                             preferred_element_type=jnp.float32) + b1_ref[...])
    out = jnp.dot(hh, w2_ref[...], preferred_element_type=jnp.float32)
    o_ref[...] = jax.nn.sigmoid(out + b2_ref[0, 0])


def _chunk_spec(ch, w):
    return pl.BlockSpec((ch, w), lambda i: (i, 0))


def _full_spec(shape):
    return pl.BlockSpec(shape, lambda i: tuple(0 for _ in shape))


def kernel(x, edge_index, batch, drug_idx, cell_idx, Wl1, bl1, Wr1, br1, g1,
           be1, Wl2, bl2, Wr2, br2, g2, be2, Wg, bg, drug_table, cell_table,
           Wh1, bh1, Wh2, bh2):
    f32 = jnp.float32
    e3 = edge_index.reshape(2, ER, 128)
    xp = jnp.pad(x, ((0, NP - N), (0, 0)))
    xo16 = jnp.concatenate(
        [xp, jnp.ones((NP, 1), f32), jnp.zeros((NP, 14), f32)], axis=1)
    batch_p = jnp.pad(batch, (0, NP - N), constant_values=-1).reshape(NP, 1)

    pa, pb = _sc_edge_agg(True, e3, xo16, xo16)

    gmat = pl.pallas_call(
        _b1_body,
        grid=(NP // CH1,),
        in_specs=[_chunk_spec(CH1, 16), _chunk_spec(CH1, 16),
                  _chunk_spec(CH1, 1)],
        out_specs=_full_spec((3, 3)),
        out_shape=jax.ShapeDtypeStruct((3, 3), f32),
    )(pa, pb, xp)

    wl = Wl1[:, 0]
    wr = Wr1[:, 0]
    c0 = bl1 + br1
    sm, sx = gmat[0, 2], gmat[1, 2]
    smm, sxx, smx = gmat[0, 0], gmat[1, 1], gmat[0, 1]
    mu = (sm * wl + sx * wr) / N + c0
    e2 = (smm * wl * wl + sxx * wr * wr + 2.0 * smx * wl * wr
          + 2.0 * c0 * (sm * wl + sx * wr)) / N + c0 * c0
    var = e2 - mu * mu
    inv = g1 / jnp.sqrt(var + 1e-5)
    cmat = jnp.stack([inv * wl, inv * wr, inv * (c0 - mu) + be1])

    h1a, h1b = pl.pallas_call(
        _b2_body,
        grid=(NP // CH1,),
        in_specs=[_chunk_spec(CH1, 16), _chunk_spec(CH1, 16),
                  _chunk_spec(CH1, 1), _full_spec((3, H))],
        out_specs=[_chunk_spec(CH1, 16), _chunk_spec(CH1, 16)],
        out_shape=[jax.ShapeDtypeStruct((NP, 16), f32)] * 2,
    )(pa, pb, xp, cmat)

    s2a, s2b = _sc_edge_agg(False, e3, h1a, h1b)

    pre2, stats = pl.pallas_call(
        _d1_body,
        grid=(NP // CH1,),
        in_specs=[_chunk_spec(CH1, 16)] * 4
        + [_chunk_spec(CH1, 16)] * 2
        + [_full_spec((H, H)), _full_spec((H, H)), _full_spec((1, H))],
        out_specs=[_chunk_spec(CH1, H), _full_spec((2, H))],
        out_shape=[jax.ShapeDtypeStruct((NP, H), f32),
                   jax.ShapeDtypeStruct((2, H), f32)],
    )(s2a, s2b, pa, pb, h1a, h1b, Wl2.T, Wr2.T, (bl2 + br2).reshape(1, H))

    m2 = stats[0] / N
    v2 = stats[1] / N - m2 * m2
    inv2 = g2 / jnp.sqrt(v2 + 1e-5)
    sc2 = inv2.reshape(1, H)
    sh2 = (be2 - m2 * inv2).reshape(1, H)

    _, den, num = pl.pallas_call(
        _d23_body,
        grid=(NP // CH2,),
        in_specs=[_chunk_spec(CH2, H), _chunk_spec(CH2, 1),
                  _full_spec((1, H)), _full_spec((1, H)),
                  _full_spec((1, H)), _full_spec((1, 1))],
        out_specs=[_full_spec((1, 1)), _full_spec((1, B)),
                   _full_spec((B, H))],
        out_shape=[jax.ShapeDtypeStruct((1, 1), f32),
                   jax.ShapeDtypeStruct((1, B), f32),
                   jax.ShapeDtypeStruct((B, H), f32)],
    )(pre2, batch_p, sc2, sh2, Wg, bg.reshape(1, 1))

    dtp = jnp.pad(drug_table, ((0, 1536 - drug_table.shape[0]), (0, 0)))
    ctp = jnp.pad(cell_table, ((0, 1024 - cell_table.shape[0]), (0, 0)))
    w2p = jnp.zeros((H, 128), f32).at[:, 0].set(Wh2[0])
    out = pl.pallas_call(
        _d4_body,
        in_specs=[_full_spec((B, H)), _full_spec((B, 1)),
                  _full_spec((B, 1)), _full_spec((B, 1)),
                  _full_spec((1536, 16)), _full_spec((1024, 16)),
                  _full_spec((2 * 16 + H, H)), _full_spec((1, H)),
                  _full_spec((H, 128)), _full_spec((1, 1))],
        out_specs=_full_spec((B, 128)),
        out_shape=jax.ShapeDtypeStruct((B, 128), f32),
        grid=(1,),
    )(num, den.reshape(B, 1), drug_idx.reshape(B, 1),
      cell_idx.reshape(B, 1), dtp, ctp, Wh1.T, bh1.reshape(1, H), w2p,
      bh2.reshape(1, 1))
    return out[:, 0]
